# Initial kernel scaffold; baseline (speedup 1.0000x reference)
import jax
import jax.numpy as jnp
from jax.experimental import pallas as pl


def kernel(x, edge_index, W1, b1, a1, W2, b2, a2):
    raise NotImplementedError("write your pallas kernel here")


# speedup vs baseline: 8.0094x; 8.0094x over previous
import functools

import jax
import jax.numpy as jnp
from jax import lax
from jax.experimental import pallas as pl
from jax.experimental.pallas import tpu as pltpu
from jax.experimental.pallas import tpu_sc as plsc

N = 10000
NPAD = 10240
NC, NS = 2, 16
RPT = NPAD // NS
CPC = ((0, 320), (320, 320))
CPB = 320
K = 128
E = 320000
CH1 = 160
CH2 = 80
JB = 16
OCP = ((0, 128), (128, 128), (256, 128), (384, 128), (512, 128))
RB = 2560

_mesh = lambda: plsc.VectorSubcoreMesh(core_axis_name="c", subcore_axis_name="s")


def _sc_deg(dstr, zeros128, ones128):
    @functools.partial(
        pl.kernel,
        out_type=jax.ShapeDtypeStruct((NC, NPAD, 128), jnp.float32),
        mesh=_mesh(),
        scratch_types=[
            pltpu.VMEM((JB, K), jnp.int32),
            pltpu.VMEM((K, 128), jnp.float32),
            pltpu.VMEM_SHARED((NPAD, 128), jnp.float32),
        ],
    )
    def k(dst_hbm, z_hbm, ones_hbm, out_hbm, dst_v, ones_v, acc_sh):
        c = lax.axis_index("c")
        s = lax.axis_index("s")
        wid = s * NC + c
        for off_h, sz in OCP:
            off = s * RPT + off_h
            pltpu.sync_copy(z_hbm.at[pl.ds(off, sz)], ones_v.at[pl.ds(0, sz)])
            pltpu.sync_copy(ones_v.at[pl.ds(0, sz)], acc_sh.at[pl.ds(off, sz)])
        pltpu.sync_copy(ones_hbm, ones_v)
        plsc.subcore_barrier()

        def outer(o, carry):
            j0 = pl.multiple_of(o * JB, JB)
            pltpu.sync_copy(dst_hbm.at[wid, pl.ds(j0, JB)], dst_v)

            def inner(j, carry2):
                pltpu.sync_copy(ones_v, acc_sh.at[dst_v.at[j]], add=True)
                return carry2

            lax.fori_loop(0, JB, inner, 0)
            return carry

        lax.fori_loop(0, CH2 // JB, outer, 0)
        plsc.subcore_barrier()
        for off_h, sz in OCP:
            off = s * RPT + off_h
            pltpu.sync_copy(acc_sh.at[pl.ds(off, sz)], ones_v.at[pl.ds(0, sz)])
            pltpu.sync_copy(ones_v.at[pl.ds(0, sz)], out_hbm.at[c, pl.ds(off, sz)])

    return k(dstr, zeros128, ones128)


def _sc_scatter_l1(g1f, srcr, dstr, zeros128):
    @functools.partial(
        pl.kernel,
        out_type=jax.ShapeDtypeStruct((NC, NPAD, 128), jnp.float32),
        mesh=_mesh(),
        scratch_types=[
            pltpu.VMEM((JB, K), jnp.int32),
            pltpu.VMEM((JB, K), jnp.int32),
            pltpu.VMEM((K, 128), jnp.float32),
            pltpu.VMEM_SHARED((NPAD, 128), jnp.float32),
            pltpu.SemaphoreType.DMA,
        ],
    )
    def k(g_hbm, src_hbm, dst_hbm, z_hbm, out_hbm,
          src_v, dst_v, rows_v, acc_sh, sem):
        c = lax.axis_index("c")
        s = lax.axis_index("s")
        for off_h, sz in OCP:
            off = s * RPT + off_h
            pltpu.sync_copy(z_hbm.at[pl.ds(off, sz)], rows_v.at[pl.ds(0, sz)])
            pltpu.sync_copy(rows_v.at[pl.ds(0, sz)], acc_sh.at[pl.ds(off, sz)])
        plsc.subcore_barrier()

        def outer(o, carry):
            j0 = pl.multiple_of(o * JB, JB)
            pltpu.sync_copy(src_hbm.at[c, s, pl.ds(j0, JB)], src_v)
            pltpu.sync_copy(dst_hbm.at[s, pl.ds(j0, JB)], dst_v)

            def inner(j, carry2):
                pltpu.async_copy(g_hbm.at[src_v.at[j]], rows_v, sem).wait()
                pltpu.sync_copy(rows_v, acc_sh.at[dst_v.at[j]], add=True)
                return carry2

            lax.fori_loop(0, JB, inner, 0)
            return carry

        lax.fori_loop(0, CH1 // JB, outer, 0)
        plsc.subcore_barrier()
        for off_h, sz in OCP:
            off = s * RPT + off_h
            pltpu.sync_copy(acc_sh.at[pl.ds(off, sz)], rows_v.at[pl.ds(0, sz)])
            pltpu.sync_copy(rows_v.at[pl.ds(0, sz)], out_hbm.at[c, pl.ds(off, sz)])

    return k(g1f, srcr, dstr, zeros128)


def _sc_scatter_l2(g2, srcr, dstr, zeros128):
    @functools.partial(
        pl.kernel,
        out_type=jax.ShapeDtypeStruct((NC, NPAD, 128), jnp.float32),
        mesh=_mesh(),
        scratch_types=[
            pltpu.VMEM((JB, K), jnp.int32),
            pltpu.VMEM((JB, K), jnp.int32),
            pltpu.VMEM((K, 128), jnp.float32),
            pltpu.VMEM_SHARED((NPAD, 128), jnp.float32),
            pltpu.SemaphoreType.DMA,
        ],
    )
    def k(g_hbm, src_hbm, dst_hbm, z_hbm, out_hbm,
          src_v, dst_v, rows_v, acc_sh, sem):
        c = lax.axis_index("c")
        s = lax.axis_index("s")
        wid = s * NC + c
        for off_h, sz in OCP:
            off = s * RPT + off_h
            pltpu.sync_copy(z_hbm.at[pl.ds(off, sz)], rows_v.at[pl.ds(0, sz)])
            pltpu.sync_copy(rows_v.at[pl.ds(0, sz)], acc_sh.at[pl.ds(off, sz)])
        plsc.subcore_barrier()

        def outer(o, carry):
            j0 = pl.multiple_of(o * JB, JB)
            pltpu.sync_copy(src_hbm.at[wid, pl.ds(j0, JB)], src_v)
            pltpu.sync_copy(dst_hbm.at[wid, pl.ds(j0, JB)], dst_v)

            def inner(j, carry2):
                pltpu.async_copy(g_hbm.at[src_v.at[j]], rows_v, sem).wait()
                pltpu.sync_copy(rows_v, acc_sh.at[dst_v.at[j]], add=True)
                return carry2

            lax.fori_loop(0, JB, inner, 0)
            return carry

        lax.fori_loop(0, CH2 // JB, outer, 0)
        plsc.subcore_barrier()
        for off_h, sz in OCP:
            off = s * RPT + off_h
            pltpu.sync_copy(acc_sh.at[pl.ds(off, sz)], rows_v.at[pl.ds(0, sz)])
            pltpu.sync_copy(rows_v.at[pl.ds(0, sz)], out_hbm.at[c, pl.ds(off, sz)])

    return k(g2, srcr, dstr, zeros128)


def _prelu(t, a_row):
    return jnp.where(t >= 0, t, t * a_row)


def _tc_g1(x_pad, W1, degp):
    def body(x_ref, w_ref, dp_ref, g1_ref, dinv_ref):
        i = pl.program_id(0)
        deg = dp_ref[0] + dp_ref[1]
        deg0 = deg[:, 0:1] + 1.0
        row = i * RB + lax.broadcasted_iota(jnp.int32, (RB, 1), 0)
        dinv = jnp.where(row < N, lax.rsqrt(deg0), 0.0)
        h = jnp.dot(x_ref[...], w_ref[...],
                    preferred_element_type=jnp.float32)
        g = h * dinv
        g1_ref[0] = g[:, :128]
        g1_ref[1] = g[:, 128:]
        dinv_ref[...] = jnp.broadcast_to(dinv, (RB, 128))

    return pl.pallas_call(
        body,
        grid=(NPAD // RB,),
        in_specs=[
            pl.BlockSpec((RB, 128), lambda i: (i, 0)),
            pl.BlockSpec((128, 256), lambda i: (0, 0)),
            pl.BlockSpec((2, RB, 128), lambda i: (0, i, 0)),
        ],
        out_specs=[
            pl.BlockSpec((2, RB, 128), lambda i: (0, i, 0)),
            pl.BlockSpec((RB, 128), lambda i: (i, 0)),
        ],
        out_shape=[
            jax.ShapeDtypeStruct((2, NPAD, 128), jnp.float32),
            jax.ShapeDtypeStruct((NPAD, 128), jnp.float32),
        ],
    )(x_pad, W1, degp)


def _tc_g2(acc1, g1p, dinvb, b1r, a1r, W2):
    def body(acc_ref, g1_ref, dinv_ref, b_ref, a_ref, w_ref, g2_ref):
        dinv = dinv_ref[...]
        a_row = a_ref[...]
        t0 = dinv * (acc_ref[0] + g1_ref[0]) + b_ref[:, :128]
        t1 = dinv * (acc_ref[1] + g1_ref[1]) + b_ref[:, 128:]
        z = jnp.concatenate([_prelu(t0, a_row), _prelu(t1, a_row)], axis=1)
        h2 = jnp.dot(z, w_ref[...], preferred_element_type=jnp.float32)
        g2_ref[...] = h2 * dinv

    return pl.pallas_call(
        body,
        grid=(NPAD // RB,),
        in_specs=[
            pl.BlockSpec((2, RB, 128), lambda i: (0, i, 0)),
            pl.BlockSpec((2, RB, 128), lambda i: (0, i, 0)),
            pl.BlockSpec((RB, 128), lambda i: (i, 0)),
            pl.BlockSpec((1, 256), lambda i: (0, 0)),
            pl.BlockSpec((1, 128), lambda i: (0, 0)),
            pl.BlockSpec((256, 128), lambda i: (0, 0)),
        ],
        out_specs=pl.BlockSpec((RB, 128), lambda i: (i, 0)),
        out_shape=jax.ShapeDtypeStruct((NPAD, 128), jnp.float32),
    )(acc1, g1p, dinvb, b1r, a1r, W2)


def _tc_out(acc2, g2, dinvb, b2r, a2r):
    def body(acc_ref, g2_ref, dinv_ref, b_ref, a_ref, o_ref):
        t = dinv_ref[...] * (acc_ref[0] + acc_ref[1] + g2_ref[...]) + b_ref[...]
        o_ref[...] = _prelu(t, a_ref[...])

    return pl.pallas_call(
        body,
        grid=(NPAD // RB,),
        in_specs=[
            pl.BlockSpec((2, RB, 128), lambda i: (0, i, 0)),
            pl.BlockSpec((RB, 128), lambda i: (i, 0)),
            pl.BlockSpec((RB, 128), lambda i: (i, 0)),
            pl.BlockSpec((1, 128), lambda i: (0, 0)),
            pl.BlockSpec((1, 128), lambda i: (0, 0)),
        ],
        out_specs=pl.BlockSpec((RB, 128), lambda i: (i, 0)),
        out_shape=jax.ShapeDtypeStruct((NPAD, 128), jnp.float32),
    )(acc2, g2, dinvb, b2r, a2r)


def kernel(x, edge_index, W1, b1, a1, W2, b2, a2):
    src = edge_index[0].astype(jnp.int32)
    dst = edge_index[1].astype(jnp.int32)

    x_pad = jnp.pad(x, ((0, NPAD - N), (0, 0)))
    srcp1 = jnp.pad(src, (0, NS * CH1 * K - E), constant_values=N).reshape(NS, CH1, K)
    srcr1 = jnp.stack([srcp1, srcp1 + NPAD])
    dstr1 = jnp.pad(dst, (0, NS * CH1 * K - E), constant_values=N).reshape(NS, CH1, K)
    srcr2 = jnp.pad(src, (0, NC * NS * CH2 * K - E), constant_values=N).reshape(NC * NS, CH2, K)
    dstr2 = jnp.pad(dst, (0, NC * NS * CH2 * K - E), constant_values=N).reshape(NC * NS, CH2, K)

    zeros128 = jnp.zeros((NPAD, 128), jnp.float32)
    ones128 = jnp.ones((K, 128), jnp.float32)

    b1r = b1.reshape(1, -1)
    b2r = b2.reshape(1, -1)
    a1r = jnp.broadcast_to(a1.reshape(1, 1), (1, 128))
    a2r = jnp.broadcast_to(a2.reshape(1, 1), (1, 128))

    degp = _sc_deg(dstr2, zeros128, ones128)
    g1p, dinvb = _tc_g1(x_pad, W1, degp)
    acc1 = _sc_scatter_l1(g1p.reshape(NC * NPAD, 128), srcr1, dstr1, zeros128)
    g2 = _tc_g2(acc1, g1p, dinvb, b1r, a1r, W2)
    acc2 = _sc_scatter_l2(g2, srcr2, dstr2, zeros128)
    out = _tc_out(acc2, g2, dinvb, b2r, a2r)
    return out[:N]


# speedup vs baseline: 8.7401x; 1.0912x over previous
import functools

import jax
import jax.numpy as jnp
from jax import lax
from jax.experimental import pallas as pl
from jax.experimental.pallas import tpu as pltpu
from jax.experimental.pallas import tpu_sc as plsc

N = 10000
NPAD = 10240
NC, NS = 2, 16
RPT = NPAD // NS
K = 128
E = 320000
CH1 = 160
CH2 = 80
JB = 16
OCP = ((0, 128), (128, 128), (256, 128), (384, 128), (512, 128))
RB = 2560

_mesh = lambda: plsc.VectorSubcoreMesh(core_axis_name="c", subcore_axis_name="s")


def _init_acc(z_hbm, bounce, acc_sh, s):
    for off_h, sz in OCP:
        off = s * RPT + off_h
        pltpu.sync_copy(z_hbm.at[pl.ds(off, sz)], bounce.at[pl.ds(0, sz)])
        pltpu.sync_copy(bounce.at[pl.ds(0, sz)], acc_sh.at[pl.ds(off, sz)])


def _copy_out(acc_sh, bounce, out_hbm, c, s):
    for off_h, sz in OCP:
        off = s * RPT + off_h
        pltpu.sync_copy(acc_sh.at[pl.ds(off, sz)], bounce.at[pl.ds(0, sz)])
        pltpu.sync_copy(bounce.at[pl.ds(0, sz)], out_hbm.at[c, pl.ds(off, sz)])


def _sc_deg(dstr, zeros128, ones128):
    @functools.partial(
        pl.kernel,
        out_type=jax.ShapeDtypeStruct((NC, NPAD, 128), jnp.float32),
        mesh=_mesh(),
        scratch_types=[
            pltpu.VMEM((JB, K), jnp.int32),
            pltpu.VMEM((K, 128), jnp.float32),
            pltpu.VMEM_SHARED((NPAD, 128), jnp.float32),
            pltpu.SemaphoreType.DMA,
        ],
    )
    def k(dst_hbm, z_hbm, ones_hbm, out_hbm, dst_v, ones_v, acc_sh, ssem):
        c = lax.axis_index("c")
        s = lax.axis_index("s")
        wid = s * NC + c
        _init_acc(z_hbm, ones_v, acc_sh, s)
        pltpu.sync_copy(ones_hbm, ones_v)
        plsc.subcore_barrier()

        def outer(o, carry):
            j0 = pl.multiple_of(o * JB, JB)
            pltpu.sync_copy(dst_hbm.at[wid, pl.ds(j0, JB)], dst_v)
            descs = [
                pltpu.async_copy(ones_v, acc_sh.at[dst_v.at[j]], ssem, add=True)
                for j in range(JB)
            ]
            for d in descs:
                d.wait()
            return carry

        lax.fori_loop(0, CH2 // JB, outer, 0)
        plsc.subcore_barrier()
        _copy_out(acc_sh, ones_v, out_hbm, c, s)

    return k(dstr, zeros128, ones128)


def _edge_pass(g_hbm, src_slab, dst_slab, acc_sh, bufs, gsems, ssems):
    gd = [None, None]
    sd = [None, None]
    gd[0] = pltpu.async_copy(g_hbm.at[src_slab.at[0]], bufs[0], gsems[0])
    for j in range(JB):
        b = j % 2
        gd[b].wait()
        if j + 1 < JB:
            if sd[1 - b] is not None:
                sd[1 - b].wait()
                sd[1 - b] = None
            gd[1 - b] = pltpu.async_copy(
                g_hbm.at[src_slab.at[j + 1]], bufs[1 - b], gsems[1 - b])
        sd[b] = pltpu.async_copy(
            bufs[b], acc_sh.at[dst_slab.at[j]], ssems[b], add=True)
    for b in range(2):
        if sd[b] is not None:
            sd[b].wait()


_EDGE_SCRATCH = lambda: [
    pltpu.VMEM((JB, K), jnp.int32),
    pltpu.VMEM((JB, K), jnp.int32),
    pltpu.VMEM((K, 128), jnp.float32),
    pltpu.VMEM((K, 128), jnp.float32),
    pltpu.VMEM_SHARED((NPAD, 128), jnp.float32),
    pltpu.SemaphoreType.DMA,
    pltpu.SemaphoreType.DMA,
    pltpu.SemaphoreType.DMA,
    pltpu.SemaphoreType.DMA,
]


def _sc_scatter_l1(g1f, srcr, dstr, zeros128):
    @functools.partial(
        pl.kernel,
        out_type=jax.ShapeDtypeStruct((NC, NPAD, 128), jnp.float32),
        mesh=_mesh(),
        scratch_types=_EDGE_SCRATCH(),
    )
    def k(g_hbm, src_hbm, dst_hbm, z_hbm, out_hbm,
          src_v, dst_v, rows_a, rows_b, acc_sh, gsa, gsb, ssa, ssb):
        c = lax.axis_index("c")
        s = lax.axis_index("s")
        _init_acc(z_hbm, rows_a, acc_sh, s)
        plsc.subcore_barrier()

        def outer(o, carry):
            j0 = pl.multiple_of(o * JB, JB)
            pltpu.sync_copy(src_hbm.at[c, s, pl.ds(j0, JB)], src_v)
            pltpu.sync_copy(dst_hbm.at[s, pl.ds(j0, JB)], dst_v)
            _edge_pass(g_hbm, src_v, dst_v, acc_sh,
                       (rows_a, rows_b), (gsa, gsb), (ssa, ssb))
            return carry

        lax.fori_loop(0, CH1 // JB, outer, 0)
        plsc.subcore_barrier()
        _copy_out(acc_sh, rows_a, out_hbm, c, s)

    return k(g1f, srcr, dstr, zeros128)


def _sc_scatter_l2(g2, srcr, dstr, zeros128):
    @functools.partial(
        pl.kernel,
        out_type=jax.ShapeDtypeStruct((NC, NPAD, 128), jnp.float32),
        mesh=_mesh(),
        scratch_types=_EDGE_SCRATCH(),
    )
    def k(g_hbm, src_hbm, dst_hbm, z_hbm, out_hbm,
          src_v, dst_v, rows_a, rows_b, acc_sh, gsa, gsb, ssa, ssb):
        c = lax.axis_index("c")
        s = lax.axis_index("s")
        wid = s * NC + c
        _init_acc(z_hbm, rows_a, acc_sh, s)
        plsc.subcore_barrier()

        def outer(o, carry):
            j0 = pl.multiple_of(o * JB, JB)
            pltpu.sync_copy(src_hbm.at[wid, pl.ds(j0, JB)], src_v)
            pltpu.sync_copy(dst_hbm.at[wid, pl.ds(j0, JB)], dst_v)
            _edge_pass(g_hbm, src_v, dst_v, acc_sh,
                       (rows_a, rows_b), (gsa, gsb), (ssa, ssb))
            return carry

        lax.fori_loop(0, CH2 // JB, outer, 0)
        plsc.subcore_barrier()
        _copy_out(acc_sh, rows_a, out_hbm, c, s)

    return k(g2, srcr, dstr, zeros128)


def _prelu(t, a_row):
    return jnp.where(t >= 0, t, t * a_row)


def _tc_g1(x_pad, W1, degp):
    def body(x_ref, w_ref, dp_ref, g1_ref, dinv_ref):
        i = pl.program_id(0)
        deg = dp_ref[0] + dp_ref[1]
        deg0 = deg[:, 0:1] + 1.0
        row = i * RB + lax.broadcasted_iota(jnp.int32, (RB, 1), 0)
        dinv = jnp.where(row < N, lax.rsqrt(deg0), 0.0)
        h = jnp.dot(x_ref[...], w_ref[...],
                    preferred_element_type=jnp.float32)
        g = h * dinv
        g1_ref[0] = g[:, :128]
        g1_ref[1] = g[:, 128:]
        dinv_ref[...] = jnp.broadcast_to(dinv, (RB, 128))

    return pl.pallas_call(
        body,
        grid=(NPAD // RB,),
        in_specs=[
            pl.BlockSpec((RB, 128), lambda i: (i, 0)),
            pl.BlockSpec((128, 256), lambda i: (0, 0)),
            pl.BlockSpec((2, RB, 128), lambda i: (0, i, 0)),
        ],
        out_specs=[
            pl.BlockSpec((2, RB, 128), lambda i: (0, i, 0)),
            pl.BlockSpec((RB, 128), lambda i: (i, 0)),
        ],
        out_shape=[
            jax.ShapeDtypeStruct((2, NPAD, 128), jnp.float32),
            jax.ShapeDtypeStruct((NPAD, 128), jnp.float32),
        ],
    )(x_pad, W1, degp)


def _tc_g2(acc1, g1p, dinvb, b1r, a1r, W2):
    def body(acc_ref, g1_ref, dinv_ref, b_ref, a_ref, w_ref, g2_ref):
        dinv = dinv_ref[...]
        a_row = a_ref[...]
        t0 = dinv * (acc_ref[0] + g1_ref[0]) + b_ref[:, :128]
        t1 = dinv * (acc_ref[1] + g1_ref[1]) + b_ref[:, 128:]
        z = jnp.concatenate([_prelu(t0, a_row), _prelu(t1, a_row)], axis=1)
        h2 = jnp.dot(z, w_ref[...], preferred_element_type=jnp.float32)
        g2_ref[...] = h2 * dinv

    return pl.pallas_call(
        body,
        grid=(NPAD // RB,),
        in_specs=[
            pl.BlockSpec((2, RB, 128), lambda i: (0, i, 0)),
            pl.BlockSpec((2, RB, 128), lambda i: (0, i, 0)),
            pl.BlockSpec((RB, 128), lambda i: (i, 0)),
            pl.BlockSpec((1, 256), lambda i: (0, 0)),
            pl.BlockSpec((1, 128), lambda i: (0, 0)),
            pl.BlockSpec((256, 128), lambda i: (0, 0)),
        ],
        out_specs=pl.BlockSpec((RB, 128), lambda i: (i, 0)),
        out_shape=jax.ShapeDtypeStruct((NPAD, 128), jnp.float32),
    )(acc1, g1p, dinvb, b1r, a1r, W2)


def _tc_out(acc2, g2, dinvb, b2r, a2r):
    def body(acc_ref, g2_ref, dinv_ref, b_ref, a_ref, o_ref):
        t = dinv_ref[...] * (acc_ref[0] + acc_ref[1] + g2_ref[...]) + b_ref[...]
        o_ref[...] = _prelu(t, a_ref[...])

    return pl.pallas_call(
        body,
        grid=(NPAD // RB,),
        in_specs=[
            pl.BlockSpec((2, RB, 128), lambda i: (0, i, 0)),
            pl.BlockSpec((RB, 128), lambda i: (i, 0)),
            pl.BlockSpec((RB, 128), lambda i: (i, 0)),
            pl.BlockSpec((1, 128), lambda i: (0, 0)),
            pl.BlockSpec((1, 128), lambda i: (0, 0)),
        ],
        out_specs=pl.BlockSpec((RB, 128), lambda i: (i, 0)),
        out_shape=jax.ShapeDtypeStruct((NPAD, 128), jnp.float32),
    )(acc2, g2, dinvb, b2r, a2r)


def kernel(x, edge_index, W1, b1, a1, W2, b2, a2):
    src = edge_index[0].astype(jnp.int32)
    dst = edge_index[1].astype(jnp.int32)

    x_pad = jnp.pad(x, ((0, NPAD - N), (0, 0)))
    srcp1 = jnp.pad(src, (0, NS * CH1 * K - E), constant_values=N).reshape(NS, CH1, K)
    srcr1 = jnp.stack([srcp1, srcp1 + NPAD])
    dstr1 = jnp.pad(dst, (0, NS * CH1 * K - E), constant_values=N).reshape(NS, CH1, K)
    srcr2 = jnp.pad(src, (0, NC * NS * CH2 * K - E), constant_values=N).reshape(NC * NS, CH2, K)
    dstr2 = jnp.pad(dst, (0, NC * NS * CH2 * K - E), constant_values=N).reshape(NC * NS, CH2, K)

    zeros128 = jnp.zeros((NPAD, 128), jnp.float32)
    ones128 = jnp.ones((K, 128), jnp.float32)

    b1r = b1.reshape(1, -1)
    b2r = b2.reshape(1, -1)
    a1r = jnp.broadcast_to(a1.reshape(1, 1), (1, 128))
    a2r = jnp.broadcast_to(a2.reshape(1, 1), (1, 128))

    degp = _sc_deg(dstr2, zeros128, ones128)
    g1p, dinvb = _tc_g1(x_pad, W1, degp)
    acc1 = _sc_scatter_l1(g1p.reshape(NC * NPAD, 128), srcr1, dstr1, zeros128)
    g2 = _tc_g2(acc1, g1p, dinvb, b1r, a1r, W2)
    acc2 = _sc_scatter_l2(g2, srcr2, dstr2, zeros128)
    out = _tc_out(acc2, g2, dinvb, b2r, a2r)
    return out[:N]


# speedup vs baseline: 19.9504x; 2.2826x over previous
import functools

import jax
import jax.numpy as jnp
from jax import lax
from jax.experimental import pallas as pl
from jax.experimental.pallas import tpu as pltpu
from jax.experimental.pallas import tpu_sc as plsc

N = 10000
NPAD = 10240
NC, NS = 2, 16
RPT = NPAD // NS
K = 128
E = 320000
CH1 = 160
CH2 = 80
JB = 16
OCP = ((0, 128), (128, 128), (256, 128), (384, 128), (512, 128))
RB = 2560

_mesh = lambda: plsc.VectorSubcoreMesh(core_axis_name="c", subcore_axis_name="s")


def _init_acc(z_hbm, bounce, acc_sh, s):
    for off_h, sz in OCP:
        off = s * RPT + off_h
        pltpu.sync_copy(z_hbm.at[pl.ds(off, sz)], bounce.at[pl.ds(0, sz)])
        pltpu.sync_copy(bounce.at[pl.ds(0, sz)], acc_sh.at[pl.ds(off, sz)])


def _copy_out(acc_sh, bounce, out_hbm, c, s):
    for off_h, sz in OCP:
        off = s * RPT + off_h
        pltpu.sync_copy(acc_sh.at[pl.ds(off, sz)], bounce.at[pl.ds(0, sz)])
        pltpu.sync_copy(bounce.at[pl.ds(0, sz)], out_hbm.at[c, pl.ds(off, sz)])


def _sc_deg(dstr, zeros128, ones128):
    @functools.partial(
        pl.kernel,
        out_type=jax.ShapeDtypeStruct((NC, NPAD, 128), jnp.float32),
        mesh=_mesh(),
        scratch_types=[
            pltpu.VMEM((JB, K), jnp.int32),
            pltpu.VMEM((K, 128), jnp.float32),
            pltpu.VMEM_SHARED((NPAD, 128), jnp.float32),
            pltpu.SemaphoreType.DMA,
        ],
    )
    def k(dst_hbm, z_hbm, ones_hbm, out_hbm, dst_v, ones_v, acc_sh, ssem):
        c = lax.axis_index("c")
        s = lax.axis_index("s")
        wid = s * NC + c
        _init_acc(z_hbm, ones_v, acc_sh, s)
        pltpu.sync_copy(ones_hbm, ones_v)
        plsc.subcore_barrier()

        def outer(o, carry):
            j0 = pl.multiple_of(o * JB, JB)
            pltpu.sync_copy(dst_hbm.at[wid, pl.ds(j0, JB)], dst_v)
            descs = [
                pltpu.async_copy(ones_v, acc_sh.at[dst_v.at[j]], ssem, add=True)
                for j in range(JB)
            ]
            for d in descs:
                d.wait()
            return carry

        lax.fori_loop(0, CH2 // JB, outer, 0)
        plsc.subcore_barrier()
        _copy_out(acc_sh, ones_v, out_hbm, c, s)

    return k(dstr, zeros128, ones128)


def _edge_pass(g_hbm, src_slab, dst_slab, acc_sh, bufs, gsems, ssems):
    gd = [None, None]
    sd = [None, None]
    gd[0] = pltpu.async_copy(g_hbm.at[src_slab.at[0]], bufs[0], gsems[0])
    for j in range(JB):
        b = j % 2
        gd[b].wait()
        if j + 1 < JB:
            if sd[1 - b] is not None:
                sd[1 - b].wait()
                sd[1 - b] = None
            gd[1 - b] = pltpu.async_copy(
                g_hbm.at[src_slab.at[j + 1]], bufs[1 - b], gsems[1 - b])
        sd[b] = pltpu.async_copy(
            bufs[b], acc_sh.at[dst_slab.at[j]], ssems[b], add=True)
    for b in range(2):
        if sd[b] is not None:
            sd[b].wait()


_EDGE_SCRATCH = lambda: [
    pltpu.VMEM((JB, K), jnp.int32),
    pltpu.VMEM((JB, K), jnp.int32),
    pltpu.VMEM((K, 128), jnp.float32),
    pltpu.VMEM((K, 128), jnp.float32),
    pltpu.VMEM_SHARED((NPAD, 128), jnp.float32),
    pltpu.SemaphoreType.DMA,
    pltpu.SemaphoreType.DMA,
    pltpu.SemaphoreType.DMA,
    pltpu.SemaphoreType.DMA,
]


def _sc_scatter_l1(g1f, srcr, dstr, zeros128):
    @functools.partial(
        pl.kernel,
        out_type=jax.ShapeDtypeStruct((NC, NPAD, 128), jnp.float32),
        mesh=_mesh(),
        scratch_types=_EDGE_SCRATCH(),
    )
    def k(g_hbm, src_hbm, dst_hbm, z_hbm, out_hbm,
          src_v, dst_v, rows_a, rows_b, acc_sh, gsa, gsb, ssa, ssb):
        c = lax.axis_index("c")
        s = lax.axis_index("s")
        _init_acc(z_hbm, rows_a, acc_sh, s)
        plsc.subcore_barrier()

        def outer(o, carry):
            j0 = pl.multiple_of(o * JB, JB)
            pltpu.sync_copy(src_hbm.at[c, s, pl.ds(j0, JB)], src_v)
            pltpu.sync_copy(dst_hbm.at[s, pl.ds(j0, JB)], dst_v)
            _edge_pass(g_hbm, src_v, dst_v, acc_sh,
                       (rows_a, rows_b), (gsa, gsb), (ssa, ssb))
            return carry

        lax.fori_loop(0, CH1 // JB, outer, 0)
        plsc.subcore_barrier()
        _copy_out(acc_sh, rows_a, out_hbm, c, s)

    return k(g1f, srcr, dstr, zeros128)


def _sc_scatter_l2(g2, srcr, dstr, zeros128):
    @functools.partial(
        pl.kernel,
        out_type=jax.ShapeDtypeStruct((NC, NPAD, 128), jnp.float32),
        mesh=_mesh(),
        scratch_types=_EDGE_SCRATCH(),
    )
    def k(g_hbm, src_hbm, dst_hbm, z_hbm, out_hbm,
          src_v, dst_v, rows_a, rows_b, acc_sh, gsa, gsb, ssa, ssb):
        c = lax.axis_index("c")
        s = lax.axis_index("s")
        wid = s * NC + c
        _init_acc(z_hbm, rows_a, acc_sh, s)
        plsc.subcore_barrier()

        def outer(o, carry):
            j0 = pl.multiple_of(o * JB, JB)
            pltpu.sync_copy(src_hbm.at[wid, pl.ds(j0, JB)], src_v)
            pltpu.sync_copy(dst_hbm.at[wid, pl.ds(j0, JB)], dst_v)
            _edge_pass(g_hbm, src_v, dst_v, acc_sh,
                       (rows_a, rows_b), (gsa, gsb), (ssa, ssb))
            return carry

        lax.fori_loop(0, CH2 // JB, outer, 0)
        plsc.subcore_barrier()
        _copy_out(acc_sh, rows_a, out_hbm, c, s)

    return k(g2, srcr, dstr, zeros128)


def _prelu(t, a_row):
    return jnp.where(t >= 0, t, t * a_row)


def _tc_g1(x_pad, W1, degp):
    def body(x_ref, w_ref, dp_ref, g1_ref, dinv_ref):
        i = pl.program_id(0)
        deg = dp_ref[0] + dp_ref[1]
        deg0 = deg[:, 0:1] + 1.0
        row = i * RB + lax.broadcasted_iota(jnp.int32, (RB, 1), 0)
        dinv = jnp.where(row < N, lax.rsqrt(deg0), 0.0)
        h = jnp.dot(x_ref[...], w_ref[...],
                    preferred_element_type=jnp.float32)
        g = h * dinv
        g1_ref[0] = g[:, :128]
        g1_ref[1] = g[:, 128:]
        dinv_ref[...] = jnp.broadcast_to(dinv, (RB, 128))

    return pl.pallas_call(
        body,
        grid=(NPAD // RB,),
        in_specs=[
            pl.BlockSpec((RB, 128), lambda i: (i, 0)),
            pl.BlockSpec((128, 256), lambda i: (0, 0)),
            pl.BlockSpec((2, RB, 128), lambda i: (0, i, 0)),
        ],
        out_specs=[
            pl.BlockSpec((2, RB, 128), lambda i: (0, i, 0)),
            pl.BlockSpec((RB, 128), lambda i: (i, 0)),
        ],
        out_shape=[
            jax.ShapeDtypeStruct((2, NPAD, 128), jnp.float32),
            jax.ShapeDtypeStruct((NPAD, 128), jnp.float32),
        ],
    )(x_pad, W1, degp)


def _tc_g2(acc1, g1p, dinvb, b1r, a1r, W2):
    def body(acc_ref, g1_ref, dinv_ref, b_ref, a_ref, w_ref, g2_ref):
        dinv = dinv_ref[...]
        a_row = a_ref[...]
        t0 = dinv * (acc_ref[0] + g1_ref[0]) + b_ref[:, :128]
        t1 = dinv * (acc_ref[1] + g1_ref[1]) + b_ref[:, 128:]
        z = jnp.concatenate([_prelu(t0, a_row), _prelu(t1, a_row)], axis=1)
        h2 = jnp.dot(z, w_ref[...], preferred_element_type=jnp.float32)
        g2_ref[...] = h2 * dinv

    return pl.pallas_call(
        body,
        grid=(NPAD // RB,),
        in_specs=[
            pl.BlockSpec((2, RB, 128), lambda i: (0, i, 0)),
            pl.BlockSpec((2, RB, 128), lambda i: (0, i, 0)),
            pl.BlockSpec((RB, 128), lambda i: (i, 0)),
            pl.BlockSpec((1, 256), lambda i: (0, 0)),
            pl.BlockSpec((1, 128), lambda i: (0, 0)),
            pl.BlockSpec((256, 128), lambda i: (0, 0)),
        ],
        out_specs=pl.BlockSpec((RB, 128), lambda i: (i, 0)),
        out_shape=jax.ShapeDtypeStruct((NPAD, 128), jnp.float32),
    )(acc1, g1p, dinvb, b1r, a1r, W2)


def _tc_out(acc2, g2, dinvb, b2r, a2r):
    def body(acc_ref, g2_ref, dinv_ref, b_ref, a_ref, o_ref):
        t = dinv_ref[...] * (acc_ref[0] + acc_ref[1] + g2_ref[...]) + b_ref[...]
        o_ref[...] = _prelu(t, a_ref[...])

    return pl.pallas_call(
        body,
        grid=(NPAD // RB,),
        in_specs=[
            pl.BlockSpec((2, RB, 128), lambda i: (0, i, 0)),
            pl.BlockSpec((RB, 128), lambda i: (i, 0)),
            pl.BlockSpec((RB, 128), lambda i: (i, 0)),
            pl.BlockSpec((1, 128), lambda i: (0, 0)),
            pl.BlockSpec((1, 128), lambda i: (0, 0)),
        ],
        out_specs=pl.BlockSpec((RB, 128), lambda i: (i, 0)),
        out_shape=jax.ShapeDtypeStruct((NPAD, 128), jnp.float32),
    )(acc2, g2, dinvb, b2r, a2r)


def kernel(x, edge_index, W1, b1, a1, W2, b2, a2):
    src = edge_index[0].astype(jnp.int32)
    dst = edge_index[1].astype(jnp.int32)

    x_pad = jnp.pad(x, ((0, NPAD - N), (0, 0)))
    EP = NS * CH1 * K - E
    padv = N + (jnp.arange(EP, dtype=jnp.int32) % (NPAD - N))
    srcp1 = jnp.concatenate([src, padv]).reshape(NS, CH1, K)
    srcr1 = jnp.stack([srcp1, srcp1 + NPAD])
    dstr1 = jnp.concatenate([dst, padv]).reshape(NS, CH1, K)
    srcr2 = jnp.concatenate([src, padv]).reshape(NC * NS, CH2, K)
    dstr2 = jnp.concatenate([dst, padv]).reshape(NC * NS, CH2, K)

    zeros128 = jnp.zeros((NPAD, 128), jnp.float32)
    ones128 = jnp.ones((K, 128), jnp.float32)

    b1r = b1.reshape(1, -1)
    b2r = b2.reshape(1, -1)
    a1r = jnp.broadcast_to(a1.reshape(1, 1), (1, 128))
    a2r = jnp.broadcast_to(a2.reshape(1, 1), (1, 128))

    degp = _sc_deg(dstr2, zeros128, ones128)
    g1p, dinvb = _tc_g1(x_pad, W1, degp)
    acc1 = _sc_scatter_l1(g1p.reshape(NC * NPAD, 128), srcr1, dstr1, zeros128)
    g2 = _tc_g2(acc1, g1p, dinvb, b1r, a1r, W2)
    acc2 = _sc_scatter_l2(g2, srcr2, dstr2, zeros128)
    out = _tc_out(acc2, g2, dinvb, b2r, a2r)
    return out[:N]


# speedup vs baseline: 21.6221x; 1.0838x over previous
import functools

import jax
import jax.numpy as jnp
from jax import lax
from jax.experimental import pallas as pl
from jax.experimental.pallas import tpu as pltpu
from jax.experimental.pallas import tpu_sc as plsc

N = 10000
NPAD = 10240
NC, NS = 2, 16
RPT = NPAD // NS
K = 128
E = 320000
CH1 = 160
CH2 = 80
JB = 32
OCP = ((0, 128), (128, 128), (256, 128), (384, 128), (512, 128))
RB = 2560

_mesh = lambda: plsc.VectorSubcoreMesh(core_axis_name="c", subcore_axis_name="s")


def _init_acc(z_hbm, bounce, acc_sh, s):
    for off_h, sz in OCP:
        off = s * RPT + off_h
        pltpu.sync_copy(z_hbm.at[pl.ds(off, sz)], bounce.at[pl.ds(0, sz)])
        pltpu.sync_copy(bounce.at[pl.ds(0, sz)], acc_sh.at[pl.ds(off, sz)])


def _copy_out(acc_sh, bounce, out_hbm, c, s):
    for off_h, sz in OCP:
        off = s * RPT + off_h
        pltpu.sync_copy(acc_sh.at[pl.ds(off, sz)], bounce.at[pl.ds(0, sz)])
        pltpu.sync_copy(bounce.at[pl.ds(0, sz)], out_hbm.at[c, pl.ds(off, sz)])


def _sc_deg(dstr, zeros128, ones128):
    @functools.partial(
        pl.kernel,
        out_type=jax.ShapeDtypeStruct((NC, NPAD, 128), jnp.float32),
        mesh=_mesh(),
        scratch_types=[
            pltpu.VMEM((JB, K), jnp.int32),
            pltpu.VMEM((K, 128), jnp.float32),
            pltpu.VMEM_SHARED((NPAD, 128), jnp.float32),
            pltpu.SemaphoreType.DMA,
        ],
    )
    def k(dst_hbm, z_hbm, ones_hbm, out_hbm, dst_v, ones_v, acc_sh, ssem):
        c = lax.axis_index("c")
        s = lax.axis_index("s")
        wid = s * NC + c
        _init_acc(z_hbm, ones_v, acc_sh, s)
        pltpu.sync_copy(ones_hbm, ones_v)
        plsc.subcore_barrier()

        def outer(o, carry):
            j0 = pl.multiple_of(o * JB, JB)
            pltpu.sync_copy(dst_hbm.at[wid, pl.ds(j0, JB)], dst_v)
            descs = [
                pltpu.async_copy(ones_v, acc_sh.at[dst_v.at[j]], ssem, add=True)
                for j in range(JB)
            ]
            for d in descs:
                d.wait()
            return carry

        lax.fori_loop(0, CH2 // JB, outer, 0)
        plsc.subcore_barrier()
        _copy_out(acc_sh, ones_v, out_hbm, c, s)

    return k(dstr, zeros128, ones128)


def _edge_pass(g_hbm, src_slab, dst_slab, acc_sh, bufs, gsems, ssems):
    gd = [None, None]
    sd = [None, None]
    gd[0] = pltpu.async_copy(g_hbm.at[src_slab.at[0]], bufs[0], gsems[0])
    for j in range(JB):
        b = j % 2
        gd[b].wait()
        if j + 1 < JB:
            if sd[1 - b] is not None:
                sd[1 - b].wait()
                sd[1 - b] = None
            gd[1 - b] = pltpu.async_copy(
                g_hbm.at[src_slab.at[j + 1]], bufs[1 - b], gsems[1 - b])
        sd[b] = pltpu.async_copy(
            bufs[b], acc_sh.at[dst_slab.at[j]], ssems[b], add=True)
    for b in range(2):
        if sd[b] is not None:
            sd[b].wait()


_EDGE_SCRATCH = lambda: [
    pltpu.VMEM((JB, K), jnp.int32),
    pltpu.VMEM((JB, K), jnp.int32),
    pltpu.VMEM((K, 128), jnp.float32),
    pltpu.VMEM((K, 128), jnp.float32),
    pltpu.VMEM_SHARED((NPAD, 128), jnp.float32),
    pltpu.SemaphoreType.DMA,
    pltpu.SemaphoreType.DMA,
    pltpu.SemaphoreType.DMA,
    pltpu.SemaphoreType.DMA,
]


def _sc_scatter_l1(g1f, srcr, dstr, zeros128):
    @functools.partial(
        pl.kernel,
        out_type=jax.ShapeDtypeStruct((NC, NPAD, 128), jnp.float32),
        mesh=_mesh(),
        scratch_types=_EDGE_SCRATCH(),
    )
    def k(g_hbm, src_hbm, dst_hbm, z_hbm, out_hbm,
          src_v, dst_v, rows_a, rows_b, acc_sh, gsa, gsb, ssa, ssb):
        c = lax.axis_index("c")
        s = lax.axis_index("s")
        _init_acc(z_hbm, rows_a, acc_sh, s)
        plsc.subcore_barrier()

        def outer(o, carry):
            j0 = pl.multiple_of(o * JB, JB)
            pltpu.sync_copy(src_hbm.at[c, s, pl.ds(j0, JB)], src_v)
            pltpu.sync_copy(dst_hbm.at[s, pl.ds(j0, JB)], dst_v)
            _edge_pass(g_hbm, src_v, dst_v, acc_sh,
                       (rows_a, rows_b), (gsa, gsb), (ssa, ssb))
            return carry

        lax.fori_loop(0, CH1 // JB, outer, 0)
        plsc.subcore_barrier()
        _copy_out(acc_sh, rows_a, out_hbm, c, s)

    return k(g1f, srcr, dstr, zeros128)


def _sc_scatter_l2(g2, srcr, dstr, zeros128):
    @functools.partial(
        pl.kernel,
        out_type=jax.ShapeDtypeStruct((NC, NPAD, 128), jnp.float32),
        mesh=_mesh(),
        scratch_types=_EDGE_SCRATCH(),
    )
    def k(g_hbm, src_hbm, dst_hbm, z_hbm, out_hbm,
          src_v, dst_v, rows_a, rows_b, acc_sh, gsa, gsb, ssa, ssb):
        c = lax.axis_index("c")
        s = lax.axis_index("s")
        wid = s * NC + c
        _init_acc(z_hbm, rows_a, acc_sh, s)
        plsc.subcore_barrier()

        def outer(o, carry):
            j0 = pl.multiple_of(o * JB, JB)
            pltpu.sync_copy(src_hbm.at[wid, pl.ds(j0, JB)], src_v)
            pltpu.sync_copy(dst_hbm.at[wid, pl.ds(j0, JB)], dst_v)
            _edge_pass(g_hbm, src_v, dst_v, acc_sh,
                       (rows_a, rows_b), (gsa, gsb), (ssa, ssb))
            return carry

        lax.fori_loop(0, CH2 // JB, outer, 0)
        plsc.subcore_barrier()
        _copy_out(acc_sh, rows_a, out_hbm, c, s)

    return k(g2, srcr, dstr, zeros128)


def _prelu(t, a_row):
    return jnp.where(t >= 0, t, t * a_row)


def _tc_mm1(x_pad, W1):
    def body(x_ref, w_ref, h_ref):
        h_ref[...] = jnp.dot(x_ref[...], w_ref[...],
                             preferred_element_type=jnp.float32)

    return pl.pallas_call(
        body,
        grid=(NPAD // RB,),
        in_specs=[
            pl.BlockSpec((RB, 128), lambda i: (i, 0)),
            pl.BlockSpec((128, 256), lambda i: (0, 0)),
        ],
        out_specs=pl.BlockSpec((RB, 256), lambda i: (i, 0)),
        out_shape=jax.ShapeDtypeStruct((NPAD, 256), jnp.float32),
    )(x_pad, W1)


def _tc_g1(h1, degp):
    def body(h_ref, dp_ref, g1_ref, dinv_ref):
        i = pl.program_id(0)
        deg = dp_ref[0] + dp_ref[1]
        deg0 = deg[:, 0:1] + 1.0
        row = i * RB + lax.broadcasted_iota(jnp.int32, (RB, 1), 0)
        dinv = jnp.where(row < N, lax.rsqrt(deg0), 0.0)
        g = h_ref[...] * dinv
        g1_ref[0] = g[:, :128]
        g1_ref[1] = g[:, 128:]
        dinv_ref[...] = jnp.broadcast_to(dinv, (RB, 128))

    return pl.pallas_call(
        body,
        grid=(NPAD // RB,),
        in_specs=[
            pl.BlockSpec((RB, 256), lambda i: (i, 0)),
            pl.BlockSpec((2, RB, 128), lambda i: (0, i, 0)),
        ],
        out_specs=[
            pl.BlockSpec((2, RB, 128), lambda i: (0, i, 0)),
            pl.BlockSpec((RB, 128), lambda i: (i, 0)),
        ],
        out_shape=[
            jax.ShapeDtypeStruct((2, NPAD, 128), jnp.float32),
            jax.ShapeDtypeStruct((NPAD, 128), jnp.float32),
        ],
    )(h1, degp)


def _tc_g2(acc1, g1p, dinvb, b1r, a1r, W2):
    def body(acc_ref, g1_ref, dinv_ref, b_ref, a_ref, w_ref, g2_ref):
        dinv = dinv_ref[...]
        a_row = a_ref[...]
        t0 = dinv * (acc_ref[0] + g1_ref[0]) + b_ref[:, :128]
        t1 = dinv * (acc_ref[1] + g1_ref[1]) + b_ref[:, 128:]
        z = jnp.concatenate([_prelu(t0, a_row), _prelu(t1, a_row)], axis=1)
        h2 = jnp.dot(z, w_ref[...], preferred_element_type=jnp.float32)
        g2_ref[...] = h2 * dinv

    return pl.pallas_call(
        body,
        grid=(NPAD // RB,),
        in_specs=[
            pl.BlockSpec((2, RB, 128), lambda i: (0, i, 0)),
            pl.BlockSpec((2, RB, 128), lambda i: (0, i, 0)),
            pl.BlockSpec((RB, 128), lambda i: (i, 0)),
            pl.BlockSpec((1, 256), lambda i: (0, 0)),
            pl.BlockSpec((1, 128), lambda i: (0, 0)),
            pl.BlockSpec((256, 128), lambda i: (0, 0)),
        ],
        out_specs=pl.BlockSpec((RB, 128), lambda i: (i, 0)),
        out_shape=jax.ShapeDtypeStruct((NPAD, 128), jnp.float32),
    )(acc1, g1p, dinvb, b1r, a1r, W2)


def _tc_out(acc2, g2, dinvb, b2r, a2r):
    def body(acc_ref, g2_ref, dinv_ref, b_ref, a_ref, o_ref):
        t = dinv_ref[...] * (acc_ref[0] + acc_ref[1] + g2_ref[...]) + b_ref[...]
        o_ref[...] = _prelu(t, a_ref[...])

    return pl.pallas_call(
        body,
        grid=(NPAD // RB,),
        in_specs=[
            pl.BlockSpec((2, RB, 128), lambda i: (0, i, 0)),
            pl.BlockSpec((RB, 128), lambda i: (i, 0)),
            pl.BlockSpec((RB, 128), lambda i: (i, 0)),
            pl.BlockSpec((1, 128), lambda i: (0, 0)),
            pl.BlockSpec((1, 128), lambda i: (0, 0)),
        ],
        out_specs=pl.BlockSpec((RB, 128), lambda i: (i, 0)),
        out_shape=jax.ShapeDtypeStruct((NPAD, 128), jnp.float32),
    )(acc2, g2, dinvb, b2r, a2r)


def kernel(x, edge_index, W1, b1, a1, W2, b2, a2):
    src = edge_index[0].astype(jnp.int32)
    dst = edge_index[1].astype(jnp.int32)

    x_pad = jnp.pad(x, ((0, NPAD - N), (0, 0)))
    EP = NS * CH1 * K - E
    padv = N + (jnp.arange(EP, dtype=jnp.int32) % (NPAD - N))
    srcp1 = jnp.concatenate([src, padv]).reshape(NS, CH1, K)
    srcr1 = jnp.stack([srcp1, srcp1 + NPAD])
    dstr1 = jnp.concatenate([dst, padv]).reshape(NS, CH1, K)
    srcr2 = jnp.concatenate([src, padv]).reshape(NC * NS, CH2, K)
    dstr2 = jnp.concatenate([dst, padv]).reshape(NC * NS, CH2, K)

    zeros128 = jnp.zeros((NPAD, 128), jnp.float32)
    ones128 = jnp.ones((K, 128), jnp.float32)

    b1r = b1.reshape(1, -1)
    b2r = b2.reshape(1, -1)
    a1r = jnp.broadcast_to(a1.reshape(1, 1), (1, 128))
    a2r = jnp.broadcast_to(a2.reshape(1, 1), (1, 128))

    h1 = _tc_mm1(x_pad, W1)
    degp = _sc_deg(dstr2, zeros128, ones128)
    g1p, dinvb = _tc_g1(h1, degp)
    acc1 = _sc_scatter_l1(g1p.reshape(NC * NPAD, 128), srcr1, dstr1, zeros128)
    g2 = _tc_g2(acc1, g1p, dinvb, b1r, a1r, W2)
    acc2 = _sc_scatter_l2(g2, srcr2, dstr2, zeros128)
    out = _tc_out(acc2, g2, dinvb, b2r, a2r)
    return out[:N]
